# baseline (speedup 1.0000x reference)
import functools

import jax
import jax.numpy as jnp
from jax.experimental import pallas as pl
from jax.experimental.pallas import tpu as pltpu

_VMEM_LIMIT = 60 * 1024 * 1024
_ROW_PAD = 256


def _round_up(x, m):
    return ((x + m - 1) // m) * m


def _pad_rows(x, n_pad):
    n = x.shape[0]
    if n == n_pad:
        return x
    return jnp.pad(x, ((0, n_pad - n),) + ((0, 0),) * (x.ndim - 1))


def _proj(x, w_ref, b_ref):
    return jnp.dot(x, w_ref[...], preferred_element_type=jnp.float32) + b_ref[...]


def _l2n(e):
    ss = jnp.sum(e * e, axis=1, keepdims=True)
    return e * jax.lax.rsqrt(jnp.maximum(ss, 1e-24))


def _fused_kernel(bm, joint_offs, wn_ref, adj_a, adj_b, entx_ref,
                  w1_ref, b1_ref, w2_ref, b2_ref,
                  img_x, rel_x, att_x, name_x, char_x, ocr_x,
                  img_w, img_b, rel_w, rel_b, att_w, att_b,
                  name_w, name_b, char_w, char_b, ocr_w, ocr_b,
                  gph_o, img_o, rel_o, att_o, name_o, char_o,
                  joint_o, ocr_o,
                  adj_c_ref, hw2_ref, modal_s_ref):
    l = pl.program_id(0)
    i = pl.program_id(1)
    row0 = pl.multiple_of(i * bm, bm)
    nh = adj_a.shape[1]

    def modal_projs():
        return (_proj(img_x[...], img_w, img_b),
                _proj(rel_x[...], rel_w, rel_b),
                _proj(att_x[...], att_w, att_b),
                _proj(name_x[...], name_w, name_b),
                _proj(char_x[...], char_w, char_b),
                _proj(ocr_x[...], ocr_w, ocr_b))

    @pl.when(l == 0)
    def _():
        a16_l = adj_a[...].astype(jnp.bfloat16)
        a16_r = adj_b[...].astype(jnp.bfloat16)
        adj_c_ref[pl.ds(row0, bm), pl.ds(0, nh)] = a16_l
        adj_c_ref[pl.ds(row0, bm), pl.ds(nh, nh)] = a16_r
        xw1 = jnp.dot(entx_ref[...], w1_ref[...],
                      preferred_element_type=jnp.float32).astype(jnp.bfloat16)
        acc = (jnp.dot(a16_l, xw1[:nh, :],
                       preferred_element_type=jnp.float32)
               + jnp.dot(a16_r, xw1[nh:, :],
                         preferred_element_type=jnp.float32))
        h = jnp.maximum(acc + b1_ref[...], 0.0)
        hw2_ref[pl.ds(row0, bm), :] = jnp.dot(
            h, w2_ref[...], preferred_element_type=jnp.float32
        ).astype(jnp.bfloat16)

        img_e, rel_e, att_e, name_e, char_e, ocr_e = modal_projs()
        moff = 0
        for e in (img_e, rel_e, att_e, name_e, char_e, ocr_e):
            modal_s_ref[pl.ds(row0, bm), moff:moff + e.shape[1]] = e
            moff += e.shape[1]

    @pl.when(l == 1)
    def _():
        a16 = adj_c_ref[pl.ds(row0, bm), :]
        gph = jnp.dot(a16, hw2_ref[...],
                      preferred_element_type=jnp.float32) + b2_ref[...]
        gph_o[...] = gph

        o_img, o_att, o_rel, o_gph, o_name, o_char, o_ocr = joint_offs
        joint_o[:, o_gph:o_gph + gph.shape[1]] = _l2n(gph) * wn_ref[3]

        moff = 0
        for o_ref, joff, wn_idx in ((img_o, o_img, 0), (rel_o, o_rel, 2),
                                    (att_o, o_att, 1), (name_o, o_name, 4),
                                    (char_o, o_char, 5), (ocr_o, o_ocr, 6)):
            d = o_ref.shape[1]
            e = modal_s_ref[pl.ds(row0, bm), moff:moff + d]
            o_ref[...] = e
            joint_o[:, joff:joff + d] = _l2n(e) * wn_ref[wn_idx]
            moff += d


def kernel(entity_emb, gc1_w, gc1_b, gc2_w, gc2_b, rel_w, rel_b, att_w, att_b,
           img_w, img_b, name_w, name_b, char_w, char_b, ocr_w, ocr_b,
           fusion_w, input_idx, adj, img_features, rel_features, att_features,
           name_features, char_features, ocr_features):
    n = adj.shape[0]
    n_pad = _round_up(max(n, _ROW_PAD), _ROW_PAD)
    bm = 256
    nh = n_pad // 2

    ent_x = _pad_rows(entity_emb[input_idx], n_pad)
    adj_p = _pad_rows(
        jnp.pad(adj, ((0, 0), (0, n_pad - n))) if n != n_pad else adj, n_pad)
    img_x = _pad_rows(img_features, n_pad)
    rel_x = _pad_rows(rel_features, n_pad)
    att_x = _pad_rows(att_features, n_pad)
    name_x = _pad_rows(name_features, n_pad)
    char_x = _pad_rows(char_features, n_pad)
    ocr_x = _pad_rows(ocr_features, n_pad)

    d_in = ent_x.shape[1]
    nhid = gc1_w.shape[1]
    nout = gc2_w.shape[1]
    b1 = gc1_b.reshape(1, -1)
    b2 = gc2_b.reshape(1, -1)

    weight_norm = jax.nn.softmax(fusion_w, axis=0)[:, 0]

    d_img = img_w.shape[1]
    d_rel = rel_w.shape[1]
    d_att = att_w.shape[1]
    d_name = name_w.shape[1]
    d_char = char_w.shape[1]
    d_ocr = ocr_w.shape[1]
    d_joint = d_img + d_att + d_rel + nout + d_name + d_char + d_ocr
    o_img = 0
    o_att = o_img + d_img
    o_rel = o_att + d_att
    o_gph = o_rel + d_rel
    o_name = o_gph + nout
    o_char = o_name + d_name
    o_ocr = o_char + d_char
    joint_offs = (o_img, o_att, o_rel, o_gph, o_name, o_char, o_ocr)

    def phase0_row_spec(d):
        return pl.BlockSpec((bm, d), lambda l, i: (i * (1 - l), 0))

    def phase1_row_spec(d):
        return pl.BlockSpec((bm, d), lambda l, i: (i * l, 0))

    def pinned(shape):
        return pl.BlockSpec(shape, lambda l, i: (0, 0))

    in_specs = [
        pl.BlockSpec(memory_space=pltpu.MemorySpace.SMEM),
        pl.BlockSpec((bm, nh), lambda l, i: (i * (1 - l), 0)),
        pl.BlockSpec((bm, nh), lambda l, i: (i * (1 - l), 1)),
        pinned((n_pad, d_in)),
        pinned((d_in, nhid)),
        pinned((1, nhid)),
        pinned((nhid, nout)),
        pinned((1, nout)),
        phase0_row_spec(img_x.shape[1]), phase0_row_spec(rel_x.shape[1]),
        phase0_row_spec(att_x.shape[1]), phase0_row_spec(name_x.shape[1]),
        phase0_row_spec(char_x.shape[1]), phase0_row_spec(ocr_x.shape[1]),
        pinned(img_w.shape), pinned((1, d_img)),
        pinned(rel_w.shape), pinned((1, d_rel)),
        pinned(att_w.shape), pinned((1, d_att)),
        pinned(name_w.shape), pinned((1, d_name)),
        pinned(char_w.shape), pinned((1, d_char)),
        pinned(ocr_w.shape), pinned((1, d_ocr)),
    ]
    out_specs = (phase1_row_spec(nout), phase1_row_spec(d_img),
                 phase1_row_spec(d_rel), phase1_row_spec(d_att),
                 phase1_row_spec(d_name), phase1_row_spec(d_char),
                 phase1_row_spec(d_joint), phase1_row_spec(d_ocr))
    out_shape = tuple(jax.ShapeDtypeStruct((n_pad, d), jnp.float32)
                      for d in (nout, d_img, d_rel, d_att, d_name, d_char,
                                d_joint, d_ocr))

    flops = (2 * n_pad * n_pad * (nhid + nout)
             + 2 * n_pad * (img_x.shape[1] * d_img + rel_x.shape[1] * d_rel
                            + att_x.shape[1] * d_att
                            + name_x.shape[1] * d_name
                            + char_x.shape[1] * d_char
                            + ocr_x.shape[1] * d_ocr))
    bytes_acc = (4 * n_pad * n_pad
                 + 4 * n_pad * (img_x.shape[1] + rel_x.shape[1]
                                + att_x.shape[1] + name_x.shape[1]
                                + char_x.shape[1] + ocr_x.shape[1])
                 + 4 * n_pad * (nout + d_img + d_rel + d_att + d_name + d_char
                                + d_joint + d_ocr))
    cost = pl.CostEstimate(flops=flops, transcendentals=7 * n_pad,
                           bytes_accessed=bytes_acc)

    outs = pl.pallas_call(
        functools.partial(_fused_kernel, bm, joint_offs),
        grid=(2, n_pad // bm),
        in_specs=in_specs,
        out_specs=out_specs,
        out_shape=out_shape,
        scratch_shapes=[pltpu.VMEM((n_pad, n_pad), jnp.bfloat16),
                        pltpu.VMEM((n_pad, nout), jnp.bfloat16),
                        pltpu.VMEM((n_pad, d_img + d_rel + d_att + d_name
                                    + d_char + d_ocr), jnp.float32)],
        compiler_params=pltpu.CompilerParams(
            dimension_semantics=("arbitrary", "arbitrary"),
            vmem_limit_bytes=_VMEM_LIMIT),
        cost_estimate=cost,
    )(weight_norm, adj_p, adj_p, ent_x, gc1_w, b1, gc2_w, b2,
      img_x, rel_x, att_x, name_x, char_x, ocr_x,
      img_w, img_b.reshape(1, -1), rel_w, rel_b.reshape(1, -1),
      att_w, att_b.reshape(1, -1), name_w, name_b.reshape(1, -1),
      char_w, char_b.reshape(1, -1), ocr_w, ocr_b.reshape(1, -1))

    gph_o, img_o, rel_o, att_o, name_o, char_o, joint_o, ocr_o = outs
    return (gph_o[:n], img_o[:n], rel_o[:n], att_o[:n], name_o[:n],
            char_o[:n], joint_o[:n], ocr_o[:n])
